# baseline (speedup 1.0000x reference)
import functools

import jax
import jax.numpy as jnp
from jax import lax
from jax.experimental import pallas as pl
from jax.experimental.pallas import tpu as pltpu
from jax.experimental.pallas import tpu_sc as plsc

_NC = 2
_NS = 16
_F = 128
_EC = 128
_RPC = 80
_DUMP = 8

_BM = 1000


def _sc_mesh():
    return plsc.VectorSubcoreMesh(
        core_axis_name="c", subcore_axis_name="s",
        num_cores=_NC, num_subcores=_NS)


@functools.lru_cache(maxsize=None)
def _make_spmm(nf, n, ep):
    ew = ep // _NS
    nch = ew // _EC
    nrow_chunks = n // _RPC
    rounds = -(-nrow_chunks // _NS)

    @functools.partial(
        pl.kernel,
        out_type=[jax.ShapeDtypeStruct((n, _F), jnp.float32)
                  for _ in range(nf)],
        mesh=_sc_mesh(),
        scratch_types=[
            pltpu.VMEM((_EC,), jnp.int32),
            pltpu.VMEM((_EC,), jnp.int32),
            pltpu.VMEM((_EC, _F), jnp.float32),
            pltpu.VMEM((_RPC, _F), jnp.float32),
            pltpu.VMEM_SHARED((n + _DUMP, _F), jnp.float32),
            pltpu.SemaphoreType.DMA,
        ],
    )
    def spmm(*refs):
        tables = refs[:nf]
        src_hbm = refs[nf]
        dst_hbm = refs[nf + 1]
        outs = refs[nf + 2:2 * nf + 2]
        isrc, idst, rows, stage, acc, sem = refs[2 * nf + 2:]
        cid = lax.axis_index("c")
        sid = lax.axis_index("s")
        ebase = sid * ew

        for fc in range(nf):
            tab = tables[fc]
            out = outs[fc]

            @pl.when(cid == (fc % _NC))
            def _(tab=tab, out=out):
                def init_body(r, carry):
                    c = r * _NS + sid

                    @pl.when(c < nrow_chunks)
                    def _():
                        off = c * _RPC
                        pltpu.sync_copy(tab.at[pl.ds(off, _RPC)], stage)
                        pltpu.sync_copy(stage, acc.at[pl.ds(off, _RPC)])
                    return carry
                lax.fori_loop(0, rounds, init_body, 0)
                plsc.subcore_barrier()

                def chunk_body(k, carry):
                    e0 = ebase + k * _EC
                    pltpu.sync_copy(src_hbm.at[pl.ds(e0, _EC)], isrc)
                    pltpu.sync_copy(dst_hbm.at[pl.ds(e0, _EC)], idst)
                    pltpu.async_copy(tab.at[isrc], rows, sem).wait()
                    pltpu.sync_copy(rows, acc.at[idst], add=True)
                    return carry
                lax.fori_loop(0, nch, chunk_body, 0)
                plsc.subcore_barrier()

                def out_body(r, carry):
                    c = r * _NS + sid

                    @pl.when(c < nrow_chunks)
                    def _():
                        off = c * _RPC
                        pltpu.sync_copy(acc.at[pl.ds(off, _RPC)], stage)
                        pltpu.sync_copy(stage, out.at[pl.ds(off, _RPC)])
                    return carry
                lax.fori_loop(0, rounds, out_body, 0)
                plsc.subcore_barrier()

    return spmm


@functools.lru_cache(maxsize=None)
def _make_deg(n, ep):
    ew = ep // (_NC * _NS)
    nch = ew // _EC
    nrow_chunks = n // _RPC
    rounds = -(-nrow_chunks // _NS)

    @functools.partial(
        pl.kernel,
        out_type=[jax.ShapeDtypeStruct((n, _F), jnp.float32)
                  for _ in range(_NC)],
        mesh=_sc_mesh(),
        scratch_types=[
            pltpu.VMEM((_EC,), jnp.int32),
            pltpu.VMEM((_EC, _F), jnp.float32),
            pltpu.VMEM((_RPC, _F), jnp.float32),
            pltpu.VMEM_SHARED((n + _DUMP, _F), jnp.float32),
        ],
    )
    def deg(tab, dst_hbm, out0, out1, idst, ones_v, stage, acc):
        cid = lax.axis_index("c")
        sid = lax.axis_index("s")
        ebase = (cid * _NS + sid) * ew
        pltpu.sync_copy(tab.at[pl.ds(0, _EC)], ones_v)

        def init_body(r, carry):
            c = r * _NS + sid

            @pl.when(c < nrow_chunks)
            def _():
                off = c * _RPC
                pltpu.sync_copy(tab.at[pl.ds(off, _RPC)], stage)
                pltpu.sync_copy(stage, acc.at[pl.ds(off, _RPC)])
            return carry
        lax.fori_loop(0, rounds, init_body, 0)
        plsc.subcore_barrier()

        def chunk_body(k, carry):
            e0 = ebase + k * _EC
            pltpu.sync_copy(dst_hbm.at[pl.ds(e0, _EC)], idst)
            pltpu.sync_copy(ones_v, acc.at[idst], add=True)
            return carry
        lax.fori_loop(0, nch, chunk_body, 0)
        plsc.subcore_barrier()

        def out_body(r, carry):
            c = r * _NS + sid

            @pl.when(c < nrow_chunks)
            def _():
                off = c * _RPC
                pltpu.sync_copy(acc.at[pl.ds(off, _RPC)], stage)

                @pl.when(cid == 0)
                def _():
                    pltpu.sync_copy(stage, out0.at[pl.ds(off, _RPC)])

                @pl.when(cid == 1)
                def _():
                    pltpu.sync_copy(stage, out1.at[pl.ds(off, _RPC)])
            return carry
        lax.fori_loop(0, rounds, out_body, 0)
        plsc.subcore_barrier()

    return deg


@functools.lru_cache(maxsize=None)
def _make_prep(n):
    def body(dega_ref, degb_ref, x_ref, dinv_ref, p0a_ref, p0b_ref):
        deg = dega_ref[:, 0:1] + degb_ref[:, 0:1] - 1.0
        dinv = lax.rsqrt(deg)
        dinv_ref[...] = dinv
        p0a_ref[...] = x_ref[:, :_F] * dinv
        p0b_ref[...] = x_ref[:, _F:] * dinv

    return pl.pallas_call(
        body,
        grid=(n // _BM,),
        in_specs=[
            pl.BlockSpec((_BM, _F), lambda i: (i, 0)),
            pl.BlockSpec((_BM, _F), lambda i: (i, 0)),
            pl.BlockSpec((_BM, 2 * _F), lambda i: (i, 0)),
        ],
        out_specs=[
            pl.BlockSpec((_BM, 1), lambda i: (i, 0)),
            pl.BlockSpec((_BM, _F), lambda i: (i, 0)),
            pl.BlockSpec((_BM, _F), lambda i: (i, 0)),
        ],
        out_shape=[
            jax.ShapeDtypeStruct((n, 1), jnp.float32),
            jax.ShapeDtypeStruct((n, _F), jnp.float32),
            jax.ShapeDtypeStruct((n, _F), jnp.float32),
        ],
    )


@functools.lru_cache(maxsize=None)
def _make_mm(n, nf_in, nf_out, scaled, use_tanh):
    k_dim = nf_in * _F
    n_dim = nf_out * _F

    def body(*refs):
        parts = refs[:nf_in]
        i = nf_in
        if scaled:
            dinv_ref = refs[i]; i += 1
            b_ref = refs[i]; i += 1
        w_ref = refs[i]; i += 1
        outs = refs[i:]
        lhs = jnp.concatenate([p[...] for p in parts], axis=1)
        res = jnp.dot(lhs, w_ref[...],
                      preferred_element_type=jnp.float32,
                      precision=lax.Precision.HIGHEST)
        if scaled:
            dinv = dinv_ref[...]
            res = res * dinv + b_ref[...]
            if use_tanh:
                res = jnp.tanh(res) * dinv
        for j in range(nf_out):
            outs[j][...] = res[:, j * _F:(j + 1) * _F]

    in_specs = [pl.BlockSpec((_BM, _F), lambda i: (i, 0))
                for _ in range(nf_in)]
    if scaled:
        in_specs.append(pl.BlockSpec((_BM, 1), lambda i: (i, 0)))
        in_specs.append(pl.BlockSpec((1, n_dim), lambda i: (0, 0)))
    in_specs.append(pl.BlockSpec((k_dim, n_dim), lambda i: (0, 0)))

    return pl.pallas_call(
        body,
        grid=(n // _BM,),
        in_specs=in_specs,
        out_specs=[pl.BlockSpec((_BM, _F), lambda i: (i, 0))
                   for _ in range(nf_out)],
        out_shape=[jax.ShapeDtypeStruct((n, _F), jnp.float32)
                   for _ in range(nf_out)],
    )


@functools.lru_cache(maxsize=None)
def _make_final(n):
    def body(sa_ref, sb_ref, dinv_ref, b_ref, out_ref):
        s = jnp.concatenate([sa_ref[...], sb_ref[...]], axis=1)
        out_ref[...] = s * dinv_ref[...] + b_ref[...]

    return pl.pallas_call(
        body,
        grid=(n // _BM,),
        in_specs=[
            pl.BlockSpec((_BM, _F), lambda i: (i, 0)),
            pl.BlockSpec((_BM, _F), lambda i: (i, 0)),
            pl.BlockSpec((_BM, 1), lambda i: (i, 0)),
            pl.BlockSpec((1, 2 * _F), lambda i: (0, 0)),
        ],
        out_specs=pl.BlockSpec((_BM, 2 * _F), lambda i: (i, 0)),
        out_shape=jax.ShapeDtypeStruct((n, 2 * _F), jnp.float32),
    )


def kernel(x, edge_index, W1, b1, W2, b2, W3, b3):
    n = x.shape[0]
    e = edge_index.shape[1]

    ep = -(-e // (_NC * _NS * _EC)) * (_NC * _NS * _EC)
    ei = edge_index.astype(jnp.int32)
    src = jnp.concatenate([ei[0], jnp.zeros((ep - e,), jnp.int32)])
    dst = jnp.concatenate([ei[1], jnp.full((ep - e,), n, jnp.int32)])

    ones_tab = jnp.ones((n, _F), dtype=jnp.float32)
    dega, degb = _make_deg(n, ep)(ones_tab, dst)

    dinv, p0a, p0b = _make_prep(n)(dega, degb, x)

    s0 = _make_spmm(2, n, ep)(p0a, p0b, src, dst)
    p1 = _make_mm(n, 2, 4, True, True)(*s0, dinv, b1.reshape(1, -1), W1)
    s1 = _make_spmm(4, n, ep)(*p1, src, dst)
    p2 = _make_mm(n, 4, 4, True, True)(*s1, dinv, b2.reshape(1, -1), W2)
    g = _make_mm(n, 4, 2, False, False)(*p2, W3)
    s2 = _make_spmm(2, n, ep)(*g, src, dst)
    out = _make_final(n)(*s2, dinv, b3.reshape(1, -1))
    return out


# speedup vs baseline: 1.7227x; 1.7227x over previous
import functools

import jax
import jax.numpy as jnp
from jax import lax
from jax.experimental import pallas as pl
from jax.experimental.pallas import tpu as pltpu
from jax.experimental.pallas import tpu_sc as plsc

_NC = 2
_NS = 16
_F = 128
_EC = 80
_RPC = 80
_DUMP = 8

_BM = 1000


def _sc_mesh():
    return plsc.VectorSubcoreMesh(
        core_axis_name="c", subcore_axis_name="s",
        num_cores=_NC, num_subcores=_NS)


@functools.lru_cache(maxsize=None)
def _make_spmm(nf, n, ep):
    ew = ep // _NS
    nch = ew // _EC
    nrow_chunks = n // _RPC
    rounds = -(-nrow_chunks // _NS)

    @functools.partial(
        pl.kernel,
        out_type=[jax.ShapeDtypeStruct((n, _F), jnp.float32)
                  for _ in range(nf)],
        mesh=_sc_mesh(),
        scratch_types=[
            pltpu.VMEM((_EC,), jnp.int32),
            pltpu.VMEM((_EC,), jnp.int32),
            pltpu.VMEM((_EC,), jnp.int32),
            pltpu.VMEM((_EC,), jnp.int32),
            pltpu.VMEM((_EC, _F), jnp.float32),
            pltpu.VMEM((_EC, _F), jnp.float32),
            pltpu.VMEM((_RPC, _F), jnp.float32),
            pltpu.VMEM_SHARED((n + _DUMP, _F), jnp.float32),
            pltpu.SemaphoreType.DMA,
            pltpu.SemaphoreType.DMA,
        ],
    )
    def spmm(*refs):
        tables = refs[:nf]
        src_hbm = refs[nf]
        dst_hbm = refs[nf + 1]
        outs = refs[nf + 2:2 * nf + 2]
        (isrca, idsta, isrcb, idstb, rowsa, rowsb, stage, acc,
         sema, semb) = refs[2 * nf + 2:]
        cid = lax.axis_index("c")
        sid = lax.axis_index("s")
        ebase = sid * ew

        for fc in range(nf):
            tab = tables[fc]
            out = outs[fc]

            @pl.when(cid == (fc % _NC))
            def _(tab=tab, out=out):
                def init_body(r, carry):
                    c = r * _NS + sid

                    @pl.when(c < nrow_chunks)
                    def _():
                        off = c * _RPC
                        pltpu.sync_copy(tab.at[pl.ds(off, _RPC)], stage)
                        pltpu.sync_copy(stage, acc.at[pl.ds(off, _RPC)])
                    return carry
                lax.fori_loop(0, rounds, init_body, 0)
                plsc.subcore_barrier()

                pltpu.sync_copy(src_hbm.at[pl.ds(ebase, _EC)], isrca)
                pltpu.sync_copy(dst_hbm.at[pl.ds(ebase, _EC)], idsta)
                pltpu.async_copy(tab.at[isrca], rowsa, sema)

                def chunk_body(k2, carry):
                    e1 = ebase + (2 * k2 + 1) * _EC
                    pltpu.sync_copy(src_hbm.at[pl.ds(e1, _EC)], isrcb)
                    pltpu.sync_copy(dst_hbm.at[pl.ds(e1, _EC)], idstb)
                    pltpu.async_copy(tab.at[isrcb], rowsb, semb)
                    pltpu.make_async_copy(tab.at[isrca], rowsa, sema).wait()
                    pltpu.sync_copy(rowsa, acc.at[idsta], add=True)

                    @pl.when(k2 + 1 < nch // 2)
                    def _():
                        e2 = ebase + (2 * k2 + 2) * _EC
                        pltpu.sync_copy(src_hbm.at[pl.ds(e2, _EC)], isrca)
                        pltpu.sync_copy(dst_hbm.at[pl.ds(e2, _EC)], idsta)
                        pltpu.async_copy(tab.at[isrca], rowsa, sema)
                    pltpu.make_async_copy(tab.at[isrcb], rowsb, semb).wait()
                    pltpu.sync_copy(rowsb, acc.at[idstb], add=True)
                    return carry
                lax.fori_loop(0, nch // 2, chunk_body, 0)
                plsc.subcore_barrier()

                def out_body(r, carry):
                    c = r * _NS + sid

                    @pl.when(c < nrow_chunks)
                    def _():
                        off = c * _RPC
                        pltpu.sync_copy(acc.at[pl.ds(off, _RPC)], stage)
                        pltpu.sync_copy(stage, out.at[pl.ds(off, _RPC)])
                    return carry
                lax.fori_loop(0, rounds, out_body, 0)
                plsc.subcore_barrier()

    return spmm


@functools.lru_cache(maxsize=None)
def _make_deg(n, ep):
    ew = ep // (_NC * _NS)
    nch = ew // _EC
    nrow_chunks = n // _RPC
    rounds = -(-nrow_chunks // _NS)

    @functools.partial(
        pl.kernel,
        out_type=[jax.ShapeDtypeStruct((n, _F), jnp.float32)
                  for _ in range(_NC)],
        mesh=_sc_mesh(),
        scratch_types=[
            pltpu.VMEM((_EC,), jnp.int32),
            pltpu.VMEM((_EC, _F), jnp.float32),
            pltpu.VMEM((_RPC, _F), jnp.float32),
            pltpu.VMEM_SHARED((n + _DUMP, _F), jnp.float32),
        ],
    )
    def deg(tab, dst_hbm, out0, out1, idst, ones_v, stage, acc):
        cid = lax.axis_index("c")
        sid = lax.axis_index("s")
        ebase = (cid * _NS + sid) * ew
        pltpu.sync_copy(tab.at[pl.ds(0, _EC)], ones_v)

        def init_body(r, carry):
            c = r * _NS + sid

            @pl.when(c < nrow_chunks)
            def _():
                off = c * _RPC
                pltpu.sync_copy(tab.at[pl.ds(off, _RPC)], stage)
                pltpu.sync_copy(stage, acc.at[pl.ds(off, _RPC)])
            return carry
        lax.fori_loop(0, rounds, init_body, 0)
        plsc.subcore_barrier()

        def chunk_body(k, carry):
            e0 = ebase + k * _EC
            pltpu.sync_copy(dst_hbm.at[pl.ds(e0, _EC)], idst)
            pltpu.sync_copy(ones_v, acc.at[idst], add=True)
            return carry
        lax.fori_loop(0, nch, chunk_body, 0)
        plsc.subcore_barrier()

        def out_body(r, carry):
            c = r * _NS + sid

            @pl.when(c < nrow_chunks)
            def _():
                off = c * _RPC
                pltpu.sync_copy(acc.at[pl.ds(off, _RPC)], stage)

                @pl.when(cid == 0)
                def _():
                    pltpu.sync_copy(stage, out0.at[pl.ds(off, _RPC)])

                @pl.when(cid == 1)
                def _():
                    pltpu.sync_copy(stage, out1.at[pl.ds(off, _RPC)])
            return carry
        lax.fori_loop(0, rounds, out_body, 0)
        plsc.subcore_barrier()

    return deg


@functools.lru_cache(maxsize=None)
def _make_prep(n):
    def body(dega_ref, degb_ref, x_ref, dinv_ref, p0a_ref, p0b_ref):
        deg = dega_ref[:, 0:1] + degb_ref[:, 0:1] - 1.0
        dinv = lax.rsqrt(deg)
        dinv_ref[...] = dinv
        p0a_ref[...] = x_ref[:, :_F] * dinv
        p0b_ref[...] = x_ref[:, _F:] * dinv

    return pl.pallas_call(
        body,
        grid=(n // _BM,),
        in_specs=[
            pl.BlockSpec((_BM, _F), lambda i: (i, 0)),
            pl.BlockSpec((_BM, _F), lambda i: (i, 0)),
            pl.BlockSpec((_BM, 2 * _F), lambda i: (i, 0)),
        ],
        out_specs=[
            pl.BlockSpec((_BM, 1), lambda i: (i, 0)),
            pl.BlockSpec((_BM, _F), lambda i: (i, 0)),
            pl.BlockSpec((_BM, _F), lambda i: (i, 0)),
        ],
        out_shape=[
            jax.ShapeDtypeStruct((n, 1), jnp.float32),
            jax.ShapeDtypeStruct((n, _F), jnp.float32),
            jax.ShapeDtypeStruct((n, _F), jnp.float32),
        ],
    )


@functools.lru_cache(maxsize=None)
def _make_mm(n, nf_in, nf_out, scaled, use_tanh):
    k_dim = nf_in * _F
    n_dim = nf_out * _F

    def body(*refs):
        parts = refs[:nf_in]
        i = nf_in
        if scaled:
            dinv_ref = refs[i]; i += 1
            b_ref = refs[i]; i += 1
        w_ref = refs[i]; i += 1
        outs = refs[i:]
        lhs = jnp.concatenate([p[...] for p in parts], axis=1)
        res = jnp.dot(lhs, w_ref[...],
                      preferred_element_type=jnp.float32,
                      precision=lax.Precision.HIGHEST)
        if scaled:
            dinv = dinv_ref[...]
            res = res * dinv + b_ref[...]
            if use_tanh:
                res = jnp.tanh(res) * dinv
        for j in range(nf_out):
            outs[j][...] = res[:, j * _F:(j + 1) * _F]

    in_specs = [pl.BlockSpec((_BM, _F), lambda i: (i, 0))
                for _ in range(nf_in)]
    if scaled:
        in_specs.append(pl.BlockSpec((_BM, 1), lambda i: (i, 0)))
        in_specs.append(pl.BlockSpec((1, n_dim), lambda i: (0, 0)))
    in_specs.append(pl.BlockSpec((k_dim, n_dim), lambda i: (0, 0)))

    return pl.pallas_call(
        body,
        grid=(n // _BM,),
        in_specs=in_specs,
        out_specs=[pl.BlockSpec((_BM, _F), lambda i: (i, 0))
                   for _ in range(nf_out)],
        out_shape=[jax.ShapeDtypeStruct((n, _F), jnp.float32)
                   for _ in range(nf_out)],
    )


@functools.lru_cache(maxsize=None)
def _make_final(n):
    def body(sa_ref, sb_ref, dinv_ref, b_ref, out_ref):
        s = jnp.concatenate([sa_ref[...], sb_ref[...]], axis=1)
        out_ref[...] = s * dinv_ref[...] + b_ref[...]

    return pl.pallas_call(
        body,
        grid=(n // _BM,),
        in_specs=[
            pl.BlockSpec((_BM, _F), lambda i: (i, 0)),
            pl.BlockSpec((_BM, _F), lambda i: (i, 0)),
            pl.BlockSpec((_BM, 1), lambda i: (i, 0)),
            pl.BlockSpec((1, 2 * _F), lambda i: (0, 0)),
        ],
        out_specs=pl.BlockSpec((_BM, 2 * _F), lambda i: (i, 0)),
        out_shape=jax.ShapeDtypeStruct((n, 2 * _F), jnp.float32),
    )


def kernel(x, edge_index, W1, b1, W2, b2, W3, b3):
    n = x.shape[0]
    e = edge_index.shape[1]

    ep = -(-e // (_NC * _NS * _EC)) * (_NC * _NS * _EC)
    ei = edge_index.astype(jnp.int32)
    src = jnp.concatenate([ei[0], jnp.zeros((ep - e,), jnp.int32)])
    dst = jnp.concatenate([ei[1], jnp.full((ep - e,), n, jnp.int32)])

    ones_tab = jnp.ones((n, _F), dtype=jnp.float32)
    dega, degb = _make_deg(n, ep)(ones_tab, dst)

    dinv, p0a, p0b = _make_prep(n)(dega, degb, x)

    s0 = _make_spmm(2, n, ep)(p0a, p0b, src, dst)
    p1 = _make_mm(n, 2, 4, True, True)(*s0, dinv, b1.reshape(1, -1), W1)
    s1 = _make_spmm(4, n, ep)(*p1, src, dst)
    p2 = _make_mm(n, 4, 4, True, True)(*s1, dinv, b2.reshape(1, -1), W2)
    g = _make_mm(n, 4, 2, False, False)(*p2, W3)
    s2 = _make_spmm(2, n, ep)(*g, src, dst)
    out = _make_final(n)(*s2, dinv, b3.reshape(1, -1))
    return out


# speedup vs baseline: 2.0036x; 1.1631x over previous
import functools

import jax
import jax.numpy as jnp
from jax import lax
from jax.experimental import pallas as pl
from jax.experimental.pallas import tpu as pltpu
from jax.experimental.pallas import tpu_sc as plsc

_NC = 2
_NS = 16
_F = 128
_EC = 80
_RPC = 80
_DUMP = 8

_BM = 1000


def _sc_mesh():
    return plsc.VectorSubcoreMesh(
        core_axis_name="c", subcore_axis_name="s",
        num_cores=_NC, num_subcores=_NS)


@functools.lru_cache(maxsize=None)
def _make_spmm(nf, n, ep):
    ew = ep // _NS
    nch = ew // _EC
    nrow_chunks = n // _RPC
    rounds = -(-nrow_chunks // _NS)

    @functools.partial(
        pl.kernel,
        out_type=[jax.ShapeDtypeStruct((n, _F), jnp.float32)
                  for _ in range(nf)],
        mesh=_sc_mesh(),
        scratch_types=[
            pltpu.VMEM((2, _EC), jnp.int32),
            pltpu.VMEM((2, _EC), jnp.int32),
            pltpu.VMEM((_EC, _F), jnp.float32),
            pltpu.VMEM((_EC, _F), jnp.float32),
            pltpu.VMEM((_RPC, _F), jnp.float32),
            pltpu.VMEM_SHARED((n + _DUMP, _F), jnp.float32),
            pltpu.SemaphoreType.DMA,
            pltpu.SemaphoreType.DMA,
            pltpu.SemaphoreType.DMA,
            pltpu.SemaphoreType.DMA,
        ],
    )
    def spmm(*refs):
        tables = refs[:nf]
        ei_hbm = refs[nf]
        outs = refs[nf + 1:2 * nf + 1]
        (ibufa, ibufb, rowsa, rowsb, stage, acc,
         gsa, gsb, ssa, ssb) = refs[2 * nf + 1:]
        cid = lax.axis_index("c")
        sid = lax.axis_index("s")
        cbase = sid * nch

        for fc in range(nf):
            tab = tables[fc]
            out = outs[fc]

            @pl.when(cid == (fc % _NC))
            def _(tab=tab, out=out):
                def init_body(r, carry):
                    c = r * _NS + sid

                    @pl.when(c < nrow_chunks)
                    def _():
                        off = c * _RPC
                        pltpu.sync_copy(tab.at[pl.ds(off, _RPC)], stage)
                        pltpu.sync_copy(stage, acc.at[pl.ds(off, _RPC)])
                    return carry
                lax.fori_loop(0, rounds, init_body, 0)
                plsc.subcore_barrier()

                pltpu.sync_copy(ei_hbm.at[cbase], ibufa)
                pltpu.async_copy(tab.at[ibufa.at[0]], rowsa, gsa)

                def chunk_body(k2, carry):
                    @pl.when(k2 > 0)
                    def _():
                        pltpu.make_async_copy(
                            rowsb, acc.at[ibufb.at[1]], ssb).wait()
                    pltpu.sync_copy(ei_hbm.at[cbase + 2 * k2 + 1], ibufb)
                    pltpu.async_copy(tab.at[ibufb.at[0]], rowsb, gsb)

                    pltpu.make_async_copy(tab.at[ibufa.at[0]], rowsa,
                                          gsa).wait()
                    pltpu.async_copy(rowsa, acc.at[ibufa.at[1]], ssa,
                                     add=True)

                    @pl.when(k2 + 1 < nch // 2)
                    def _():
                        pltpu.make_async_copy(
                            rowsa, acc.at[ibufa.at[1]], ssa).wait()
                        pltpu.sync_copy(ei_hbm.at[cbase + 2 * k2 + 2], ibufa)
                        pltpu.async_copy(tab.at[ibufa.at[0]], rowsa, gsa)

                    pltpu.make_async_copy(tab.at[ibufb.at[0]], rowsb,
                                          gsb).wait()
                    pltpu.async_copy(rowsb, acc.at[ibufb.at[1]], ssb,
                                     add=True)
                    return carry
                lax.fori_loop(0, nch // 2, chunk_body, 0)
                pltpu.make_async_copy(rowsa, acc.at[ibufa.at[1]], ssa).wait()
                pltpu.make_async_copy(rowsb, acc.at[ibufb.at[1]], ssb).wait()
                plsc.subcore_barrier()

                def out_body(r, carry):
                    c = r * _NS + sid

                    @pl.when(c < nrow_chunks)
                    def _():
                        off = c * _RPC
                        pltpu.sync_copy(acc.at[pl.ds(off, _RPC)], stage)
                        pltpu.sync_copy(stage, out.at[pl.ds(off, _RPC)])
                    return carry
                lax.fori_loop(0, rounds, out_body, 0)
                plsc.subcore_barrier()

    return spmm


@functools.lru_cache(maxsize=None)
def _make_deg(n, ep):
    ew = ep // (_NC * _NS)
    nch = ew // _EC
    nrow_chunks = n // _RPC
    rounds = -(-nrow_chunks // _NS)

    @functools.partial(
        pl.kernel,
        out_type=[jax.ShapeDtypeStruct((n, _F), jnp.float32)
                  for _ in range(_NC)],
        mesh=_sc_mesh(),
        scratch_types=[
            pltpu.VMEM((_EC,), jnp.int32),
            pltpu.VMEM((_EC, _F), jnp.float32),
            pltpu.VMEM((_RPC, _F), jnp.float32),
            pltpu.VMEM_SHARED((n + _DUMP, _F), jnp.float32),
        ],
    )
    def deg(tab, dst_hbm, out0, out1, idst, ones_v, stage, acc):
        cid = lax.axis_index("c")
        sid = lax.axis_index("s")
        ebase = (cid * _NS + sid) * ew
        pltpu.sync_copy(tab.at[pl.ds(0, _EC)], ones_v)

        def init_body(r, carry):
            c = r * _NS + sid

            @pl.when(c < nrow_chunks)
            def _():
                off = c * _RPC
                pltpu.sync_copy(tab.at[pl.ds(off, _RPC)], stage)
                pltpu.sync_copy(stage, acc.at[pl.ds(off, _RPC)])
            return carry
        lax.fori_loop(0, rounds, init_body, 0)
        plsc.subcore_barrier()

        def chunk_body(k, carry):
            e0 = ebase + k * _EC
            pltpu.sync_copy(dst_hbm.at[pl.ds(e0, _EC)], idst)
            pltpu.sync_copy(ones_v, acc.at[idst], add=True)
            return carry
        lax.fori_loop(0, nch, chunk_body, 0)
        plsc.subcore_barrier()

        def out_body(r, carry):
            c = r * _NS + sid

            @pl.when(c < nrow_chunks)
            def _():
                off = c * _RPC
                pltpu.sync_copy(acc.at[pl.ds(off, _RPC)], stage)

                @pl.when(cid == 0)
                def _():
                    pltpu.sync_copy(stage, out0.at[pl.ds(off, _RPC)])

                @pl.when(cid == 1)
                def _():
                    pltpu.sync_copy(stage, out1.at[pl.ds(off, _RPC)])
            return carry
        lax.fori_loop(0, rounds, out_body, 0)
        plsc.subcore_barrier()

    return deg


@functools.lru_cache(maxsize=None)
def _make_prep(n):
    def body(dega_ref, degb_ref, x_ref, dinv_ref, p0a_ref, p0b_ref):
        deg = dega_ref[:, 0:1] + degb_ref[:, 0:1] - 1.0
        dinv = lax.rsqrt(deg)
        dinv_ref[...] = dinv
        p0a_ref[...] = x_ref[:, :_F] * dinv
        p0b_ref[...] = x_ref[:, _F:] * dinv

    return pl.pallas_call(
        body,
        grid=(n // _BM,),
        in_specs=[
            pl.BlockSpec((_BM, _F), lambda i: (i, 0)),
            pl.BlockSpec((_BM, _F), lambda i: (i, 0)),
            pl.BlockSpec((_BM, 2 * _F), lambda i: (i, 0)),
        ],
        out_specs=[
            pl.BlockSpec((_BM, 1), lambda i: (i, 0)),
            pl.BlockSpec((_BM, _F), lambda i: (i, 0)),
            pl.BlockSpec((_BM, _F), lambda i: (i, 0)),
        ],
        out_shape=[
            jax.ShapeDtypeStruct((n, 1), jnp.float32),
            jax.ShapeDtypeStruct((n, _F), jnp.float32),
            jax.ShapeDtypeStruct((n, _F), jnp.float32),
        ],
    )


@functools.lru_cache(maxsize=None)
def _make_mm(n, nf_in, nf_out, scaled, use_tanh):
    k_dim = nf_in * _F
    n_dim = nf_out * _F

    def body(*refs):
        parts = refs[:nf_in]
        i = nf_in
        if scaled:
            dinv_ref = refs[i]; i += 1
            b_ref = refs[i]; i += 1
        w_ref = refs[i]; i += 1
        outs = refs[i:]
        lhs = jnp.concatenate([p[...] for p in parts], axis=1)
        res = jnp.dot(lhs, w_ref[...],
                      preferred_element_type=jnp.float32,
                      precision=lax.Precision.HIGHEST)
        if scaled:
            dinv = dinv_ref[...]
            res = res * dinv + b_ref[...]
            if use_tanh:
                res = jnp.tanh(res) * dinv
        for j in range(nf_out):
            outs[j][...] = res[:, j * _F:(j + 1) * _F]

    in_specs = [pl.BlockSpec((_BM, _F), lambda i: (i, 0))
                for _ in range(nf_in)]
    if scaled:
        in_specs.append(pl.BlockSpec((_BM, 1), lambda i: (i, 0)))
        in_specs.append(pl.BlockSpec((1, n_dim), lambda i: (0, 0)))
    in_specs.append(pl.BlockSpec((k_dim, n_dim), lambda i: (0, 0)))

    return pl.pallas_call(
        body,
        grid=(n // _BM,),
        in_specs=in_specs,
        out_specs=[pl.BlockSpec((_BM, _F), lambda i: (i, 0))
                   for _ in range(nf_out)],
        out_shape=[jax.ShapeDtypeStruct((n, _F), jnp.float32)
                   for _ in range(nf_out)],
    )


@functools.lru_cache(maxsize=None)
def _make_final(n):
    def body(sa_ref, sb_ref, dinv_ref, b_ref, out_ref):
        s = jnp.concatenate([sa_ref[...], sb_ref[...]], axis=1)
        out_ref[...] = s * dinv_ref[...] + b_ref[...]

    return pl.pallas_call(
        body,
        grid=(n // _BM,),
        in_specs=[
            pl.BlockSpec((_BM, _F), lambda i: (i, 0)),
            pl.BlockSpec((_BM, _F), lambda i: (i, 0)),
            pl.BlockSpec((_BM, 1), lambda i: (i, 0)),
            pl.BlockSpec((1, 2 * _F), lambda i: (0, 0)),
        ],
        out_specs=pl.BlockSpec((_BM, 2 * _F), lambda i: (i, 0)),
        out_shape=jax.ShapeDtypeStruct((n, 2 * _F), jnp.float32),
    )


def kernel(x, edge_index, W1, b1, W2, b2, W3, b3):
    n = x.shape[0]
    e = edge_index.shape[1]

    ep = -(-e // (_NC * _NS * _EC)) * (_NC * _NS * _EC)
    ei = edge_index.astype(jnp.int32)
    pad = jnp.concatenate([
        jnp.zeros((1, ep - e), jnp.int32),
        jnp.full((1, ep - e), n, jnp.int32),
    ], axis=0)
    ei = jnp.concatenate([ei, pad], axis=1)
    dst = ei[1]
    ei = jnp.stack([ei[0].reshape(-1, _EC), ei[1].reshape(-1, _EC)], axis=1)

    ones_tab = jnp.ones((n, _F), dtype=jnp.float32)
    dega, degb = _make_deg(n, ep)(ones_tab, dst)

    dinv, p0a, p0b = _make_prep(n)(dega, degb, x)

    s0 = _make_spmm(2, n, ep)(p0a, p0b, ei)
    p1 = _make_mm(n, 2, 4, True, True)(*s0, dinv, b1.reshape(1, -1), W1)
    s1 = _make_spmm(4, n, ep)(*p1, ei)
    p2 = _make_mm(n, 4, 4, True, True)(*s1, dinv, b2.reshape(1, -1), W2)
    g = _make_mm(n, 4, 2, False, False)(*p2, W3)
    s2 = _make_spmm(2, n, ep)(*g, ei)
    out = _make_final(n)(*s2, dinv, b3.reshape(1, -1))
    return out


# speedup vs baseline: 2.1924x; 1.0942x over previous
import functools

import jax
import jax.numpy as jnp
from jax import lax
from jax.experimental import pallas as pl
from jax.experimental.pallas import tpu as pltpu
from jax.experimental.pallas import tpu_sc as plsc

_NC = 2
_NS = 16
_F = 128
_EC = 120
_RPC = 80
_DUMP = 8

_BM = 1000


def _sc_mesh():
    return plsc.VectorSubcoreMesh(
        core_axis_name="c", subcore_axis_name="s",
        num_cores=_NC, num_subcores=_NS)


@functools.lru_cache(maxsize=None)
def _make_spmm(nf, n, ep):
    ew = ep // _NS
    nch = ew // _EC
    nrow_chunks = n // _RPC
    rounds = -(-nrow_chunks // _NS)

    @functools.partial(
        pl.kernel,
        out_type=[jax.ShapeDtypeStruct((n, _F), jnp.float32)
                  for _ in range(nf)],
        mesh=_sc_mesh(),
        scratch_types=[
            pltpu.VMEM((2, _EC), jnp.int32),
            pltpu.VMEM((2, _EC), jnp.int32),
            pltpu.VMEM((_EC, _F), jnp.float32),
            pltpu.VMEM((_EC, _F), jnp.float32),
            pltpu.VMEM((_RPC, _F), jnp.float32),
            pltpu.VMEM_SHARED((n + _DUMP, _F), jnp.float32),
            pltpu.SemaphoreType.DMA,
            pltpu.SemaphoreType.DMA,
            pltpu.SemaphoreType.DMA,
            pltpu.SemaphoreType.DMA,
        ],
    )
    def spmm(*refs):
        tables = refs[:nf]
        ei_hbm = refs[nf]
        outs = refs[nf + 1:2 * nf + 1]
        (ibufa, ibufb, rowsa, rowsb, stage, acc,
         gsa, gsb, ssa, ssb) = refs[2 * nf + 1:]
        cid = lax.axis_index("c")
        sid = lax.axis_index("s")
        cbase = sid * nch

        for fc in range(nf):
            tab = tables[fc]
            out = outs[fc]

            @pl.when(cid == (fc % _NC))
            def _(tab=tab, out=out):
                def init_body(r, carry):
                    c = r * _NS + sid

                    @pl.when(c < nrow_chunks)
                    def _():
                        off = c * _RPC
                        pltpu.sync_copy(tab.at[pl.ds(off, _RPC)], stage)
                        pltpu.sync_copy(stage, acc.at[pl.ds(off, _RPC)])
                    return carry
                lax.fori_loop(0, rounds, init_body, 0)
                plsc.subcore_barrier()

                pltpu.sync_copy(ei_hbm.at[cbase], ibufa)
                pltpu.async_copy(tab.at[ibufa.at[0]], rowsa, gsa)

                def chunk_body(k2, carry):
                    @pl.when(k2 > 0)
                    def _():
                        pltpu.make_async_copy(
                            rowsb, acc.at[ibufb.at[1]], ssb).wait()
                    pltpu.sync_copy(ei_hbm.at[cbase + 2 * k2 + 1], ibufb)
                    pltpu.async_copy(tab.at[ibufb.at[0]], rowsb, gsb)

                    pltpu.make_async_copy(tab.at[ibufa.at[0]], rowsa,
                                          gsa).wait()
                    pltpu.async_copy(rowsa, acc.at[ibufa.at[1]], ssa,
                                     add=True)

                    @pl.when(k2 + 1 < nch // 2)
                    def _():
                        pltpu.make_async_copy(
                            rowsa, acc.at[ibufa.at[1]], ssa).wait()
                        pltpu.sync_copy(ei_hbm.at[cbase + 2 * k2 + 2], ibufa)
                        pltpu.async_copy(tab.at[ibufa.at[0]], rowsa, gsa)

                    pltpu.make_async_copy(tab.at[ibufb.at[0]], rowsb,
                                          gsb).wait()
                    pltpu.async_copy(rowsb, acc.at[ibufb.at[1]], ssb,
                                     add=True)
                    return carry
                lax.fori_loop(0, nch // 2, chunk_body, 0)
                pltpu.make_async_copy(rowsa, acc.at[ibufa.at[1]], ssa).wait()
                pltpu.make_async_copy(rowsb, acc.at[ibufb.at[1]], ssb).wait()
                plsc.subcore_barrier()

                def out_body(r, carry):
                    c = r * _NS + sid

                    @pl.when(c < nrow_chunks)
                    def _():
                        off = c * _RPC
                        pltpu.sync_copy(acc.at[pl.ds(off, _RPC)], stage)
                        pltpu.sync_copy(stage, out.at[pl.ds(off, _RPC)])
                    return carry
                lax.fori_loop(0, rounds, out_body, 0)
                plsc.subcore_barrier()

    return spmm


@functools.lru_cache(maxsize=None)
def _make_deg(n, ep):
    ew = ep // (_NC * _NS)
    nch = ew // _EC
    nrow_chunks = n // _RPC
    rounds = -(-nrow_chunks // _NS)

    @functools.partial(
        pl.kernel,
        out_type=[jax.ShapeDtypeStruct((n, _F), jnp.float32)
                  for _ in range(_NC)],
        mesh=_sc_mesh(),
        scratch_types=[
            pltpu.VMEM((_EC,), jnp.int32),
            pltpu.VMEM((_EC, _F), jnp.float32),
            pltpu.VMEM((_RPC, _F), jnp.float32),
            pltpu.VMEM_SHARED((n + _DUMP, _F), jnp.float32),
        ],
    )
    def deg(tab, dst_hbm, out0, out1, idst, ones_v, stage, acc):
        cid = lax.axis_index("c")
        sid = lax.axis_index("s")
        ebase = (cid * _NS + sid) * ew
        pltpu.sync_copy(tab.at[pl.ds(0, _EC)], ones_v)

        def init_body(r, carry):
            c = r * _NS + sid

            @pl.when(c < nrow_chunks)
            def _():
                off = c * _RPC
                pltpu.sync_copy(tab.at[pl.ds(off, _RPC)], stage)
                pltpu.sync_copy(stage, acc.at[pl.ds(off, _RPC)])
            return carry
        lax.fori_loop(0, rounds, init_body, 0)
        plsc.subcore_barrier()

        def chunk_body(k, carry):
            e0 = ebase + k * _EC
            pltpu.sync_copy(dst_hbm.at[pl.ds(e0, _EC)], idst)
            pltpu.sync_copy(ones_v, acc.at[idst], add=True)
            return carry
        lax.fori_loop(0, nch, chunk_body, 0)
        plsc.subcore_barrier()

        def out_body(r, carry):
            c = r * _NS + sid

            @pl.when(c < nrow_chunks)
            def _():
                off = c * _RPC
                pltpu.sync_copy(acc.at[pl.ds(off, _RPC)], stage)

                @pl.when(cid == 0)
                def _():
                    pltpu.sync_copy(stage, out0.at[pl.ds(off, _RPC)])

                @pl.when(cid == 1)
                def _():
                    pltpu.sync_copy(stage, out1.at[pl.ds(off, _RPC)])
            return carry
        lax.fori_loop(0, rounds, out_body, 0)
        plsc.subcore_barrier()

    return deg


@functools.lru_cache(maxsize=None)
def _make_prep(n):
    def body(dega_ref, degb_ref, x_ref, dinv_ref, p0a_ref, p0b_ref):
        deg = dega_ref[:, 0:1] + degb_ref[:, 0:1] - 1.0
        dinv = lax.rsqrt(deg)
        dinv_ref[...] = dinv
        p0a_ref[...] = x_ref[:, :_F] * dinv
        p0b_ref[...] = x_ref[:, _F:] * dinv

    return pl.pallas_call(
        body,
        grid=(n // _BM,),
        in_specs=[
            pl.BlockSpec((_BM, _F), lambda i: (i, 0)),
            pl.BlockSpec((_BM, _F), lambda i: (i, 0)),
            pl.BlockSpec((_BM, 2 * _F), lambda i: (i, 0)),
        ],
        out_specs=[
            pl.BlockSpec((_BM, 1), lambda i: (i, 0)),
            pl.BlockSpec((_BM, _F), lambda i: (i, 0)),
            pl.BlockSpec((_BM, _F), lambda i: (i, 0)),
        ],
        out_shape=[
            jax.ShapeDtypeStruct((n, 1), jnp.float32),
            jax.ShapeDtypeStruct((n, _F), jnp.float32),
            jax.ShapeDtypeStruct((n, _F), jnp.float32),
        ],
    )


@functools.lru_cache(maxsize=None)
def _make_mm(n, nf_in, nf_out, scaled, use_tanh):
    k_dim = nf_in * _F
    n_dim = nf_out * _F

    def body(*refs):
        parts = refs[:nf_in]
        i = nf_in
        if scaled:
            dinv_ref = refs[i]; i += 1
            b_ref = refs[i]; i += 1
        w_ref = refs[i]; i += 1
        outs = refs[i:]
        lhs = jnp.concatenate([p[...] for p in parts], axis=1)
        res = jnp.dot(lhs, w_ref[...],
                      preferred_element_type=jnp.float32,
                      precision=lax.Precision.HIGHEST)
        if scaled:
            dinv = dinv_ref[...]
            res = res * dinv + b_ref[...]
            if use_tanh:
                res = jnp.tanh(res) * dinv
        for j in range(nf_out):
            outs[j][...] = res[:, j * _F:(j + 1) * _F]

    in_specs = [pl.BlockSpec((_BM, _F), lambda i: (i, 0))
                for _ in range(nf_in)]
    if scaled:
        in_specs.append(pl.BlockSpec((_BM, 1), lambda i: (i, 0)))
        in_specs.append(pl.BlockSpec((1, n_dim), lambda i: (0, 0)))
    in_specs.append(pl.BlockSpec((k_dim, n_dim), lambda i: (0, 0)))

    return pl.pallas_call(
        body,
        grid=(n // _BM,),
        in_specs=in_specs,
        out_specs=[pl.BlockSpec((_BM, _F), lambda i: (i, 0))
                   for _ in range(nf_out)],
        out_shape=[jax.ShapeDtypeStruct((n, _F), jnp.float32)
                   for _ in range(nf_out)],
    )


@functools.lru_cache(maxsize=None)
def _make_final(n):
    def body(sa_ref, sb_ref, dinv_ref, b_ref, out_ref):
        s = jnp.concatenate([sa_ref[...], sb_ref[...]], axis=1)
        out_ref[...] = s * dinv_ref[...] + b_ref[...]

    return pl.pallas_call(
        body,
        grid=(n // _BM,),
        in_specs=[
            pl.BlockSpec((_BM, _F), lambda i: (i, 0)),
            pl.BlockSpec((_BM, _F), lambda i: (i, 0)),
            pl.BlockSpec((_BM, 1), lambda i: (i, 0)),
            pl.BlockSpec((1, 2 * _F), lambda i: (0, 0)),
        ],
        out_specs=pl.BlockSpec((_BM, 2 * _F), lambda i: (i, 0)),
        out_shape=jax.ShapeDtypeStruct((n, 2 * _F), jnp.float32),
    )


def kernel(x, edge_index, W1, b1, W2, b2, W3, b3):
    n = x.shape[0]
    e = edge_index.shape[1]

    ep = -(-e // (_NC * _NS * _EC)) * (_NC * _NS * _EC)
    ei = edge_index.astype(jnp.int32)
    pad = jnp.concatenate([
        jnp.zeros((1, ep - e), jnp.int32),
        jnp.full((1, ep - e), n, jnp.int32),
    ], axis=0)
    ei = jnp.concatenate([ei, pad], axis=1)
    dst = ei[1]
    ei = jnp.stack([ei[0].reshape(-1, _EC), ei[1].reshape(-1, _EC)], axis=1)

    ones_tab = jnp.ones((n, _F), dtype=jnp.float32)
    dega, degb = _make_deg(n, ep)(ones_tab, dst)

    dinv, p0a, p0b = _make_prep(n)(dega, degb, x)

    s0 = _make_spmm(2, n, ep)(p0a, p0b, ei)
    p1 = _make_mm(n, 2, 4, True, True)(*s0, dinv, b1.reshape(1, -1), W1)
    s1 = _make_spmm(4, n, ep)(*p1, ei)
    p2 = _make_mm(n, 4, 4, True, True)(*s1, dinv, b2.reshape(1, -1), W2)
    g = _make_mm(n, 4, 2, False, False)(*p2, W3)
    s2 = _make_spmm(2, n, ep)(*g, ei)
    out = _make_final(n)(*s2, dinv, b3.reshape(1, -1))
    return out
